# baseline (speedup 1.0000x reference)
import functools

import jax
import jax.numpy as jnp
from jax import lax
from jax.experimental import pallas as pl
from jax.experimental.pallas import tpu as pltpu
from jax.experimental.pallas import tpu_sc as plsc

NC = 2
NS = 16
L = 16
NW = NC * NS
BATCH = 16384
BPW = BATCH // NW
NCHUNK = 4
CHUNK = BPW // NCHUNK
DIM = 64
ROW = 2 * DIM


def _body(users_hbm, items_hbm, user_hbm, item_hbm, out_hbm,
          ud_v, id_v, ugidx_v, vgidx_v, uhalf_v, vhalf_v,
          urows_v, vrows_v, out_v, sem0, sem1):
    cid = lax.axis_index("c")
    sid = lax.axis_index("s")
    wid = sid * NC + cid
    base = wid * BPW

    pltpu.sync_copy(users_hbm.at[pl.ds(base, BPW)], ud_v)
    pltpu.sync_copy(items_hbm.at[pl.ds(base, BPW)], id_v)

    iota = lax.iota(jnp.int32, L)

    for j in range(NCHUNK):
        urow = ugidx_v.at[j]
        irow = vgidx_v.at[j]
        for k in range(CHUNK // L):
            off = j * CHUNK + k * L
            u16 = ud_v[pl.ds(off, L)]
            i16 = id_v[pl.ds(off, L)]
            urow[pl.ds(k * L, L)] = ((u16 >> 14) << 13) | (u16 & 8191)
            irow[pl.ds(k * L, L)] = ((i16 >> 14) << 13) | (i16 & 8191)
            uhalf_v[pl.ds(off, L)] = ((u16 >> 13) & 1) << 6
            vhalf_v[pl.ds(off, L)] = ((i16 >> 13) & 1) << 6

    sems = (sem0, sem1)

    def fire(j):
        b = j % 2
        return (
            pltpu.async_copy(user_hbm.at[ugidx_v.at[j]], urows_v.at[b],
                             sems[b]),
            pltpu.async_copy(item_hbm.at[vgidx_v.at[j]], vrows_v.at[b],
                             sems[b]),
        )

    copies = [None] * NCHUNK
    copies[0] = fire(0)
    for j in range(NCHUNK):
        if j + 1 < NCHUNK:
            copies[j + 1] = fire(j + 1)
        copies[j][0].wait()
        copies[j][1].wait()

        jb = j % 2
        jsplat = jnp.full((L,), jb, jnp.int32)

        def grp(g, _, j=j, jsplat=jsplat):
            rows = iota + g * L
            uh = uhalf_v[pl.ds(j * CHUNK + g * L, L)]
            vh = vhalf_v[pl.ds(j * CHUNK + g * L, L)]
            def dblk(d0, accs):
                accs = list(accs)
                for t in range(DIM // 2):
                    didx = (iota + (d0 * (DIM // 2) + t)) & (DIM - 1)
                    uu = plsc.load_gather(urows_v, [jsplat, rows, uh + didx])
                    vv = plsc.load_gather(vrows_v, [jsplat, rows, vh + didx])
                    accs[t % 4] = accs[t % 4] + uu * vv
                return tuple(accs)

            accs = lax.fori_loop(
                0, 2, dblk,
                tuple(jnp.zeros((L,), jnp.float32) for _ in range(4)))
            acc = (accs[0] + accs[1]) + (accs[2] + accs[3])
            res = 1.0 / (1.0 + jnp.exp(-acc))
            out_v[pl.ds(j * CHUNK + g * L, L)] = res
            return 0

        lax.fori_loop(0, CHUNK // L, grp, 0)

    pltpu.sync_copy(out_v, out_hbm.at[pl.ds(base, BPW)])


CB = 16384
NBLK = (100000 + CB - 1) // CB


def _convert_body(dt_ref, ut_ref, it_ref, us_ref, is_ref, u2_ref, i2_ref):
    h = CB // 2
    eye = jax.lax.broadcasted_iota(jnp.int32, (DIM, DIM), 0)
    eye = (eye == jax.lax.broadcasted_iota(jnp.int32, (DIM, DIM), 1))
    eye = eye.astype(jnp.float32)
    for src, dst in ((ut_ref, u2_ref), (it_ref, i2_ref)):
        y = jax.lax.dot_general(
            src[...], eye, (((0,), (0,)), ((), ())),
            preferred_element_type=jnp.float32)
        dst[:, 0:DIM] = y[0:h, :]
        dst[:, DIM:2 * DIM] = y[h:CB, :]

    @pl.when(pl.program_id(0) == 0)
    def _():
        us_ref[...] = dt_ref[0, :]
        is_ref[...] = dt_ref[1, :]


def _convert(dt, ut, it):
    in_spec = pl.BlockSpec((DIM, CB), lambda t: (0, t))
    out_spec = pl.BlockSpec((CB // 2, 2 * DIM), lambda t: (t, 0))
    return pl.pallas_call(
        _convert_body,
        grid=(NBLK,),
        in_specs=[pl.BlockSpec((2, BATCH), lambda t: (0, 0)),
                  in_spec, in_spec],
        out_specs=[pl.BlockSpec((BATCH,), lambda t: (0,)),
                   pl.BlockSpec((BATCH,), lambda t: (0,)),
                   out_spec, out_spec],
        out_shape=[jax.ShapeDtypeStruct((BATCH,), jnp.int32)] * 2
        + [jax.ShapeDtypeStruct((NBLK * CB // 2, 2 * DIM),
                                jnp.float32)] * 2,
        compiler_params=pltpu.CompilerParams(
            fuse_transposed_lhs_in_matmul=True),
    )(dt, ut, it)


@jax.jit
def kernel(data, user_embeds, item_embeds):
    mesh = plsc.VectorSubcoreMesh(core_axis_name="c", subcore_axis_name="s")
    f = functools.partial(
        pl.kernel,
        out_type=jax.ShapeDtypeStruct((BATCH,), jnp.float32),
        mesh=mesh,
        scratch_types=[
            pltpu.VMEM((BPW,), jnp.int32),
            pltpu.VMEM((BPW,), jnp.int32),
            pltpu.VMEM((NCHUNK, CHUNK), jnp.int32),
            pltpu.VMEM((NCHUNK, CHUNK), jnp.int32),
            pltpu.VMEM((BPW,), jnp.int32),
            pltpu.VMEM((BPW,), jnp.int32),
            pltpu.VMEM((2, CHUNK, ROW), jnp.float32),
            pltpu.VMEM((2, CHUNK, ROW), jnp.float32),
            pltpu.VMEM((BPW,), jnp.float32),
            pltpu.SemaphoreType.DMA,
            pltpu.SemaphoreType.DMA,
        ],
        compiler_params=pltpu.CompilerParams(
            needs_layout_passes=False, use_tc_tiling_on_sc=True),
    )(_body)
    us, its, u2, i2 = _convert(data.T, user_embeds.T, item_embeds.T)
    return f(us, its, u2, i2)


# speedup vs baseline: 1.2061x; 1.2061x over previous
import functools

import jax
import jax.numpy as jnp
from jax import lax
from jax.experimental import pallas as pl
from jax.experimental.pallas import tpu as pltpu
from jax.experimental.pallas import tpu_sc as plsc

NC = 2
NS = 16
L = 16
NW = NC * NS
BATCH = 16384
BPW = BATCH // NW
NCHUNK = 4
CHUNK = BPW // NCHUNK
DIM = 64
ROW = 2 * DIM


def _body(users_hbm, items_hbm, user_hbm, item_hbm, out_hbm,
          ud_v, id_v, ugidx_v, vgidx_v, uhalf_v, vhalf_v,
          urows_v, vrows_v, out_v, sem0, sem1):
    cid = lax.axis_index("c")
    sid = lax.axis_index("s")
    wid = sid * NC + cid
    base = wid * BPW

    pltpu.sync_copy(users_hbm.at[pl.ds(base, BPW)], ud_v)
    pltpu.sync_copy(items_hbm.at[pl.ds(base, BPW)], id_v)

    iota = lax.iota(jnp.int32, L)

    for j in range(NCHUNK):
        urow = ugidx_v.at[j]
        irow = vgidx_v.at[j]
        for k in range(CHUNK // L):
            off = j * CHUNK + k * L
            u16 = ud_v[pl.ds(off, L)]
            i16 = id_v[pl.ds(off, L)]
            urow[pl.ds(k * L, L)] = ((u16 >> 13) << 12) | (u16 & 4095)
            irow[pl.ds(k * L, L)] = ((i16 >> 13) << 12) | (i16 & 4095)
            uhalf_v[pl.ds(off, L)] = ((u16 >> 12) & 1) << 6
            vhalf_v[pl.ds(off, L)] = ((i16 >> 12) & 1) << 6

    sems = (sem0, sem1)

    def fire(j):
        b = j % 2
        return (
            pltpu.async_copy(user_hbm.at[ugidx_v.at[j]], urows_v.at[b],
                             sems[b]),
            pltpu.async_copy(item_hbm.at[vgidx_v.at[j]], vrows_v.at[b],
                             sems[b]),
        )

    copies = [None] * NCHUNK
    copies[0] = fire(0)
    for j in range(NCHUNK):
        if j + 1 < NCHUNK:
            copies[j + 1] = fire(j + 1)
        copies[j][0].wait()
        copies[j][1].wait()

        jb = j % 2
        jsplat = jnp.full((L,), jb, jnp.int32)

        def grp(g, _, j=j, jsplat=jsplat):
            rows = iota + g * L
            uh = uhalf_v[pl.ds(j * CHUNK + g * L, L)]
            vh = vhalf_v[pl.ds(j * CHUNK + g * L, L)]
            def dblk(d0, accs):
                accs = list(accs)
                for t in range(DIM // 2):
                    didx = (iota + (d0 * (DIM // 2) + t)) & (DIM - 1)
                    uu = plsc.load_gather(urows_v, [jsplat, rows, uh + didx])
                    vv = plsc.load_gather(vrows_v, [jsplat, rows, vh + didx])
                    accs[t % 4] = accs[t % 4] + uu * vv
                return tuple(accs)

            accs = lax.fori_loop(
                0, 2, dblk,
                tuple(jnp.zeros((L,), jnp.float32) for _ in range(4)))
            acc = (accs[0] + accs[1]) + (accs[2] + accs[3])
            res = 1.0 / (1.0 + jnp.exp(-acc))
            out_v[pl.ds(j * CHUNK + g * L, L)] = res
            return 0

        lax.fori_loop(0, CHUNK // L, grp, 0)

    pltpu.sync_copy(out_v, out_hbm.at[pl.ds(base, BPW)])


CB = 8192
NBLK = (100000 + CB - 1) // CB


def _convert_body(dt_ref, ut_ref, it_ref, us_ref, is_ref, u2_ref, i2_ref):
    h = CB // 2
    eye = jax.lax.broadcasted_iota(jnp.int32, (2 * DIM, 2 * DIM), 0)
    eye = (eye == jax.lax.broadcasted_iota(jnp.int32, (2 * DIM, 2 * DIM), 1))
    eye = eye.astype(jnp.float32)
    x = jnp.concatenate([ut_ref[...], it_ref[...]], axis=0)
    y = jax.lax.dot_general(
        x, eye, (((0,), (0,)), ((), ())),
        preferred_element_type=jnp.float32)
    u2_ref[:, 0:DIM] = y[0:h, 0:DIM]
    u2_ref[:, DIM:2 * DIM] = y[h:CB, 0:DIM]
    i2_ref[:, 0:DIM] = y[0:h, DIM:2 * DIM]
    i2_ref[:, DIM:2 * DIM] = y[h:CB, DIM:2 * DIM]

    @pl.when(pl.program_id(0) == 0)
    def _():
        us_ref[...] = dt_ref[0, :]
        is_ref[...] = dt_ref[1, :]


def _convert(dt, ut, it):
    in_spec = pl.BlockSpec((DIM, CB), lambda t: (0, t))
    out_spec = pl.BlockSpec((CB // 2, 2 * DIM), lambda t: (t, 0))
    return pl.pallas_call(
        _convert_body,
        grid=(NBLK,),
        in_specs=[pl.BlockSpec((2, BATCH), lambda t: (0, 0)),
                  in_spec, in_spec],
        out_specs=[pl.BlockSpec((BATCH,), lambda t: (0,)),
                   pl.BlockSpec((BATCH,), lambda t: (0,)),
                   out_spec, out_spec],
        out_shape=[jax.ShapeDtypeStruct((BATCH,), jnp.int32)] * 2
        + [jax.ShapeDtypeStruct((NBLK * CB // 2, 2 * DIM),
                                jnp.float32)] * 2,
        compiler_params=pltpu.CompilerParams(
            fuse_transposed_lhs_in_matmul=True),
    )(dt, ut, it)


@jax.jit
def kernel(data, user_embeds, item_embeds):
    mesh = plsc.VectorSubcoreMesh(core_axis_name="c", subcore_axis_name="s")
    f = functools.partial(
        pl.kernel,
        out_type=jax.ShapeDtypeStruct((BATCH,), jnp.float32),
        mesh=mesh,
        scratch_types=[
            pltpu.VMEM((BPW,), jnp.int32),
            pltpu.VMEM((BPW,), jnp.int32),
            pltpu.VMEM((NCHUNK, CHUNK), jnp.int32),
            pltpu.VMEM((NCHUNK, CHUNK), jnp.int32),
            pltpu.VMEM((BPW,), jnp.int32),
            pltpu.VMEM((BPW,), jnp.int32),
            pltpu.VMEM((2, CHUNK, ROW), jnp.float32),
            pltpu.VMEM((2, CHUNK, ROW), jnp.float32),
            pltpu.VMEM((BPW,), jnp.float32),
            pltpu.SemaphoreType.DMA,
            pltpu.SemaphoreType.DMA,
        ],
        compiler_params=pltpu.CompilerParams(
            needs_layout_passes=False, use_tc_tiling_on_sc=True),
    )(_body)
    us, its, u2, i2 = _convert(data.T, user_embeds.T, item_embeds.T)
    return f(us, its, u2, i2)
